# Initial kernel scaffold; baseline (speedup 1.0000x reference)
import jax
import jax.numpy as jnp
from jax.experimental import pallas as pl


def kernel(inputs, embedding_weight):
    raise NotImplementedError("write your pallas kernel here")


# speedup vs baseline: 1.1154x; 1.1154x over previous
import functools

import jax
import jax.numpy as jnp
from jax import lax
from jax.experimental import pallas as pl
from jax.experimental.pallas import tpu as pltpu
from jax.experimental.pallas import tpu_sc as plsc

_T, _BCOL = 200, 4096
_V, _D = 1_000_000, 32
_B = _T * _BCOL

_info = plsc.get_sparse_core_info()
_NC, _NS = _info.num_cores, _info.num_subcores
_NW = _NC * _NS
_B_PER_W = _B // _NW
_CHUNK = 1600
_NCHUNK = _B_PER_W // _CHUNK

_mesh = plsc.VectorSubcoreMesh(core_axis_name="c", subcore_axis_name="s")


@functools.partial(
    pl.kernel,
    mesh=_mesh,
    out_type=jax.ShapeDtypeStruct((_B, _D), jnp.float32),
    scratch_types=[
        pltpu.VMEM((_B_PER_W,), jnp.int32),
        pltpu.VMEM((_CHUNK, _D), jnp.float32),
        pltpu.SemaphoreType.DMA,
    ],
    compiler_params=pltpu.CompilerParams(use_tc_tiling_on_sc=False),
)
def _emb_lookup(idx_hbm, table_hbm, out_hbm, idx_v, rows_v, sem):
    wid = lax.axis_index("s") * _NC + lax.axis_index("c")
    base = wid * _B_PER_W
    pltpu.sync_copy(idx_hbm.at[pl.ds(base, _B_PER_W)], idx_v)
    for c in range(_NCHUNK):
        off = c * _CHUNK
        pltpu.async_copy(
            table_hbm.at[idx_v.at[pl.ds(off, _CHUNK)]], rows_v, sem
        ).wait()
        pltpu.sync_copy(rows_v, out_hbm.at[pl.ds(base + off, _CHUNK)])


def kernel(inputs, embedding_weight):
    idx = inputs.reshape(-1).astype(jnp.int32)
    out = _emb_lookup(idx, embedding_weight)
    return out.reshape(_T, _BCOL, _D)


# speedup vs baseline: 1.1258x; 1.0093x over previous
import functools

import jax
import jax.numpy as jnp
from jax import lax
from jax.experimental import pallas as pl
from jax.experimental.pallas import tpu as pltpu
from jax.experimental.pallas import tpu_sc as plsc

_T, _BCOL = 200, 4096
_V, _D = 1_000_000, 32
_B = _T * _BCOL

_info = plsc.get_sparse_core_info()
_NC, _NS = _info.num_cores, _info.num_subcores
_NW = _NC * _NS
_B_PER_W = _B // _NW
_CHUNK = 1024
_NCHUNK = _B_PER_W // _CHUNK
_NBUF = 3

_mesh = plsc.VectorSubcoreMesh(core_axis_name="c", subcore_axis_name="s")


@functools.partial(
    pl.kernel,
    mesh=_mesh,
    out_type=jax.ShapeDtypeStruct((_B, _D), jnp.float32),
    scratch_types=[
        pltpu.VMEM((_B_PER_W,), jnp.int32),
        pltpu.VMEM((_NBUF, _CHUNK, _D), jnp.float32),
        pltpu.SemaphoreType.DMA((_NBUF,)),
        pltpu.SemaphoreType.DMA((_NBUF,)),
    ],
    compiler_params=pltpu.CompilerParams(use_tc_tiling_on_sc=False),
)
def _emb_lookup(idx_hbm, table_hbm, out_hbm, idx_v, rows_v, gsem, ssem):
    wid = lax.axis_index("s") * _NC + lax.axis_index("c")
    base = wid * _B_PER_W
    pltpu.sync_copy(idx_hbm.at[pl.ds(base, _B_PER_W)], idx_v)

    def gather(c):
        b = c % _NBUF
        return pltpu.async_copy(
            table_hbm.at[idx_v.at[pl.ds(c * _CHUNK, _CHUNK)]],
            rows_v.at[b],
            gsem.at[b],
        )

    def store(c):
        b = c % _NBUF
        return pltpu.async_copy(
            rows_v.at[b],
            out_hbm.at[pl.ds(base + c * _CHUNK, _CHUNK)],
            ssem.at[b],
        )

    gathers = [None] * _NCHUNK
    stores = [None] * _NCHUNK
    gathers[0] = gather(0)
    for c in range(_NCHUNK):
        if c + 1 < _NCHUNK:
            if c + 1 >= _NBUF:
                stores[c + 1 - _NBUF].wait()
            gathers[c + 1] = gather(c + 1)
        gathers[c].wait()
        stores[c] = store(c)
    for c in range(max(0, _NCHUNK - _NBUF + 1), _NCHUNK):
        stores[c].wait()


def kernel(inputs, embedding_weight):
    idx = inputs.reshape(-1).astype(jnp.int32)
    out = _emb_lookup(idx, embedding_weight)
    return out.reshape(_T, _BCOL, _D)
